# baseline (speedup 1.0000x reference)
import jax
import jax.numpy as jnp
from jax import lax
from jax.experimental import pallas as pl
from jax.experimental.pallas import tpu as pltpu
from jax.experimental.pallas import tpu_sc as plsc

N_NODES = 10000
N_EDGES = 320000
NC, NS = 2, 16
NW = NC * NS
CHUNK = 128
E_PAD = 327680
NCH_A = E_PAD // NS // CHUNK
NCH_B = E_PAD // NW // CHUNK
N_PAD = 10240
SLAB = N_PAD // NS
CNTW = 16
DC = 64

_MESH = plsc.VectorSubcoreMesh(
    core_axis_name="c", subcore_axis_name="s", num_cores=NC, num_subcores=NS)
_SC_PARAMS = pltpu.CompilerParams(use_tc_tiling_on_sc=False)


def _pipeline(table, idx_s, idx_d, rows0, rows1, acc_sh, sem0, sem1, nchunk,
              extra=None):
  def gstart(g, buf, sem):
    pltpu.async_copy(table.at[idx_s.at[g]], buf, sem)

  def gwait(buf, sem):
    pltpu.make_async_copy(table.at[idx_s.at[0]], buf, sem).wait()

  def scat(g, buf):
    pltpu.sync_copy(buf, acc_sh.at[idx_d.at[g]], add=True)
    if extra is not None:
      extra(g)

  gstart(0, rows0, sem0)

  def step(g, carry):
    gstart(2 * g + 1, rows1, sem1)
    gwait(rows0, sem0)
    scat(2 * g, rows0)
    gstart(2 * g + 2, rows0, sem0)
    gwait(rows1, sem1)
    scat(2 * g + 1, rows1)
    return carry

  if nchunk % 2:
    lax.fori_loop(0, (nchunk - 1) // 2, step, 0)
    gwait(rows0, sem0)
    scat(nchunk - 1, rows0)
  else:
    lax.fori_loop(0, nchunk // 2 - 1, step, 0)
    gstart(nchunk - 1, rows1, sem1)
    gwait(rows0, sem0)
    scat(nchunk - 2, rows0)
    gwait(rows1, sem1)
    scat(nchunk - 1, rows1)


def _zero_acc(rows0, acc_sh, s):
  zv = jnp.zeros((16,), jnp.float32)
  vpr = DC // 16
  shift = vpr.bit_length() - 1

  def zstore(i, carry):
    rows0[i >> shift, pl.ds((i & (vpr - 1)) * 16, 16)] = zv
    return carry

  lax.fori_loop(0, CHUNK * vpr, zstore, 0)
  for t in range(SLAB // CHUNK):
    pltpu.sync_copy(rows0, acc_sh.at[pl.ds(s * SLAB + t * CHUNK, CHUNK)])


def _make_seg_colsplit(with_cnt):
  out_type = [jax.ShapeDtypeStruct((NC, N_PAD, DC), jnp.float32)]
  scratch = [
      pltpu.VMEM((NCH_A, CHUNK), jnp.int32),
      pltpu.VMEM((NCH_A, CHUNK), jnp.int32),
      pltpu.VMEM((CHUNK, DC), jnp.float32),
      pltpu.VMEM((CHUNK, DC), jnp.float32),
      pltpu.VMEM_SHARED((N_PAD, DC), jnp.float32),
      pltpu.SemaphoreType.DMA,
      pltpu.SemaphoreType.DMA,
  ]
  if with_cnt:
    out_type.append(jax.ShapeDtypeStruct((NC, N_PAD, CNTW), jnp.float32))
    scratch += [
        pltpu.VMEM((CHUNK, CNTW), jnp.float32),
        pltpu.VMEM((CHUNK, CNTW), jnp.float32),
        pltpu.VMEM_SHARED((N_PAD, CNTW), jnp.float32),
    ]

  def body(*refs):
    if with_cnt:
      (table, srcw, dstw, ones_hbm, out, cnt_out, idx_s, idx_d,
       rows0, rows1, acc_sh, sem0, sem1, ones_v, zbuf, cnt_sh) = refs
    else:
      (table, srcw, dstw, out,
       idx_s, idx_d, rows0, rows1, acc_sh, sem0, sem1) = refs
    c = lax.axis_index("c")
    s = lax.axis_index("s")
    slab = pl.ds(s * SLAB, SLAB)

    pltpu.sync_copy(srcw.at[c, s], idx_s)
    pltpu.sync_copy(dstw.at[s], idx_d)
    _zero_acc(rows0, acc_sh, s)
    extra = None
    if with_cnt:
      pltpu.sync_copy(ones_hbm, ones_v)
      zc = jnp.zeros((16,), jnp.float32)

      def czero(i, carry):
        zbuf[i, pl.ds(0, 16)] = zc
        return carry

      lax.fori_loop(0, CHUNK, czero, 0)
      for t in range(SLAB // CHUNK):
        pltpu.sync_copy(zbuf, cnt_sh.at[pl.ds(s * SLAB + t * CHUNK, CHUNK)])

      def extra(g):
        pltpu.sync_copy(ones_v, cnt_sh.at[idx_d.at[g]], add=True)

    plsc.subcore_barrier()

    _pipeline(table, idx_s, idx_d, rows0, rows1, acc_sh, sem0, sem1, NCH_A,
              extra=extra)

    plsc.subcore_barrier()
    pltpu.sync_copy(acc_sh.at[slab], out.at[c, slab])
    if with_cnt:
      pltpu.sync_copy(cnt_sh.at[slab], cnt_out.at[c, slab])

  return pl.kernel(body, out_type=out_type, mesh=_MESH, scratch_types=scratch,
                   compiler_params=_SC_PARAMS,
                   name="seg_colsplit" + ("_cnt" if with_cnt else ""))


def _make_seg_rowsplit():
  out_type = [jax.ShapeDtypeStruct((NC, N_PAD, DC), jnp.float32)]
  scratch = [
      pltpu.VMEM((NCH_B, CHUNK), jnp.int32),
      pltpu.VMEM((NCH_B, CHUNK), jnp.int32),
      pltpu.VMEM((CHUNK, DC), jnp.float32),
      pltpu.VMEM((CHUNK, DC), jnp.float32),
      pltpu.VMEM_SHARED((N_PAD, DC), jnp.float32),
      pltpu.SemaphoreType.DMA,
      pltpu.SemaphoreType.DMA,
  ]

  def body(table, srcw, dstw, out, idx_s, idx_d, rows0, rows1,
           acc_sh, sem0, sem1):
    c = lax.axis_index("c")
    s = lax.axis_index("s")
    wid = c * NS + s
    slab = pl.ds(s * SLAB, SLAB)

    pltpu.sync_copy(srcw.at[wid], idx_s)
    pltpu.sync_copy(dstw.at[wid], idx_d)
    _zero_acc(rows0, acc_sh, s)
    plsc.subcore_barrier()

    _pipeline(table, idx_s, idx_d, rows0, rows1, acc_sh, sem0, sem1, NCH_B)

    plsc.subcore_barrier()
    pltpu.sync_copy(acc_sh.at[slab], out.at[c, slab])

  return pl.kernel(body, out_type=out_type, mesh=_MESH, scratch_types=scratch,
                   compiler_params=_SC_PARAMS, name="seg_rowsplit")


_seg_col_cnt = _make_seg_colsplit(True)
_seg_col = _make_seg_colsplit(False)
_seg_row = _make_seg_rowsplit()


def _inv_deg(cntp_ref):
  cnt = cntp_ref[0, :, 0:1]
  return 1.0 / jnp.maximum(cnt, 1.0)


def _dot_t(a, w):
  return lax.dot_general(a, w, (((1,), (1,)), ((), ())),
                         preferred_element_type=jnp.float32)


_NB = 1000


def _tc1_body(aggp, cntp, x, w1l, b1l, w1r, h1):
  agg = jnp.concatenate([aggp[0], aggp[1]], axis=1) * _inv_deg(cntp)
  h = _dot_t(agg, w1l[...]) + b1l[...] + _dot_t(x[...], w1r[...])
  h1[...] = jnp.maximum(h, 0.0)


def _tc2_body(aggp, cntp, h1, w2l, b2l, w2r, w3l, h2, z):
  agg = jnp.concatenate([aggp[0], aggp[1]], axis=1) * _inv_deg(cntp)
  h = _dot_t(agg, w2l[...]) + b2l[...] + _dot_t(h1[...], w2r[...])
  h = jnp.maximum(h, 0.0)
  h2[...] = h
  z[...] = _dot_t(h, w3l[...])


def _tc3_body(aggp, cntp, h2, w3r, b3l, out):
  agg = (aggp[0] + aggp[1]) * _inv_deg(cntp)
  out[...] = agg + b3l[...] + _dot_t(h2[...], w3r[...])


def _row_spec(d):
  return pl.BlockSpec((_NB, d), lambda i: (i, 0))


def _part_spec(d):
  return pl.BlockSpec((NC, _NB, d), lambda i: (0, i, 0))


def _full_spec(shape):
  return pl.BlockSpec(shape, lambda i: tuple(0 for _ in shape))


_GRID = N_NODES // _NB

_tc1 = pl.pallas_call(
    _tc1_body,
    grid=(_GRID,),
    in_specs=[_part_spec(DC), _part_spec(CNTW), _row_spec(128),
              _full_spec((128, 128)), _full_spec((1, 128)),
              _full_spec((128, 128))],
    out_specs=_row_spec(128),
    out_shape=jax.ShapeDtypeStruct((N_NODES, 128), jnp.float32),
)

_tc2 = pl.pallas_call(
    _tc2_body,
    grid=(_GRID,),
    in_specs=[_part_spec(DC), _part_spec(CNTW), _row_spec(128),
              _full_spec((256, 128)), _full_spec((1, 256)),
              _full_spec((256, 128)), _full_spec((64, 256))],
    out_specs=[_row_spec(256), _row_spec(64)],
    out_shape=[jax.ShapeDtypeStruct((N_NODES, 256), jnp.float32),
               jax.ShapeDtypeStruct((N_NODES, 64), jnp.float32)],
)

_tc3 = pl.pallas_call(
    _tc3_body,
    grid=(_GRID,),
    in_specs=[_part_spec(64), _part_spec(CNTW), _row_spec(256),
              _full_spec((64, 256)), _full_spec((1, 64))],
    out_specs=_row_spec(64),
    out_shape=jax.ShapeDtypeStruct((N_NODES, 64), jnp.float32),
)


@jax.jit
def kernel(x, edge_index, W1l, b1l, W1r, W2l, b2l, W2r, W3l, b3l, W3r):
  src = edge_index[0].astype(jnp.int32)
  dst = edge_index[1].astype(jnp.int32)
  npad = E_PAD - N_EDGES
  srcp = jnp.concatenate([src, jnp.zeros((npad,), jnp.int32)])
  dstp = jnp.concatenate([dst, jnp.full((npad,), N_PAD - 1, jnp.int32)])
  srcA = jnp.stack([2 * srcp, 2 * srcp + 1]).reshape(NC, NS, NCH_A, CHUNK)
  dstA = dstp.reshape(NS, NCH_A, CHUNK)
  srcB = srcp.reshape(NW, NCH_B, CHUNK)
  dstB = dstp.reshape(NW, NCH_B, CHUNK)
  ones = jnp.ones((CHUNK, CNTW), jnp.float32)

  x2 = x.reshape(2 * N_NODES, DC)
  aggp1, cntp = _seg_col_cnt(x2, srcA, dstA, ones)
  h1 = _tc1(aggp1, cntp, x, W1l, b1l.reshape(1, -1), W1r)
  aggp2, = _seg_col(h1.reshape(2 * N_NODES, DC), srcA, dstA)
  h2, z = _tc2(aggp2, cntp, h1, W2l, b2l.reshape(1, -1), W2r, W3l)
  aggp3, = _seg_row(z, srcB, dstB)
  return _tc3(aggp3, cntp, h2, W3r, b3l.reshape(1, -1))


# speedup vs baseline: 1.0111x; 1.0111x over previous
import jax
import jax.numpy as jnp
from jax import lax
from jax.experimental import pallas as pl
from jax.experimental.pallas import tpu as pltpu
from jax.experimental.pallas import tpu_sc as plsc

N_NODES = 10000
N_EDGES = 320000
NC, NS = 2, 16
NW = NC * NS
CHUNK = 128
E_PAD = 327680
NCH_A = E_PAD // NS // CHUNK
NCH_B = E_PAD // NW // CHUNK
N_PAD = 10240
SLAB = N_PAD // NS
CNTW = 16
DC = 64

_MESH = plsc.VectorSubcoreMesh(
    core_axis_name="c", subcore_axis_name="s", num_cores=NC, num_subcores=NS)
_SC_PARAMS = pltpu.CompilerParams(use_tc_tiling_on_sc=False)


def _pipeline(table, idx_s, idx_d, rows0, rows1, acc_sh, sem0, sem1, nchunk,
              extra=None):
  def gstart(g, buf, sem):
    pltpu.async_copy(table.at[idx_s.at[g]], buf, sem)

  def gwait(buf, sem):
    pltpu.make_async_copy(table.at[idx_s.at[0]], buf, sem).wait()

  def scat(g, buf):
    pltpu.sync_copy(buf, acc_sh.at[idx_d.at[g]], add=True)
    if extra is not None:
      extra(g)

  gstart(0, rows0, sem0)

  def step(g, carry):
    gstart(2 * g + 1, rows1, sem1)
    gwait(rows0, sem0)
    scat(2 * g, rows0)
    gstart(2 * g + 2, rows0, sem0)
    gwait(rows1, sem1)
    scat(2 * g + 1, rows1)
    return carry

  if nchunk % 2:
    lax.fori_loop(0, (nchunk - 1) // 2, step, 0)
    gwait(rows0, sem0)
    scat(nchunk - 1, rows0)
  else:
    lax.fori_loop(0, nchunk // 2 - 1, step, 0)
    gstart(nchunk - 1, rows1, sem1)
    gwait(rows0, sem0)
    scat(nchunk - 2, rows0)
    gwait(rows1, sem1)
    scat(nchunk - 1, rows1)


def _zero_acc(rows0, acc_sh, s):
  zv = jnp.zeros((16,), jnp.float32)
  vpr = DC // 16
  shift = vpr.bit_length() - 1

  def zstore(i, carry):
    rows0[i >> shift, pl.ds((i & (vpr - 1)) * 16, 16)] = zv
    return carry

  lax.fori_loop(0, CHUNK * vpr, zstore, 0)
  for t in range(SLAB // CHUNK):
    pltpu.sync_copy(rows0, acc_sh.at[pl.ds(s * SLAB + t * CHUNK, CHUNK)])


def _make_seg_colsplit(with_cnt):
  out_type = [jax.ShapeDtypeStruct((NC, N_PAD, DC), jnp.float32)]
  scratch = [
      pltpu.VMEM((NCH_A, CHUNK), jnp.int32),
      pltpu.VMEM((NCH_A, CHUNK), jnp.int32),
      pltpu.VMEM((CHUNK, DC), jnp.float32),
      pltpu.VMEM((CHUNK, DC), jnp.float32),
      pltpu.VMEM_SHARED((N_PAD, DC), jnp.float32),
      pltpu.SemaphoreType.DMA,
      pltpu.SemaphoreType.DMA,
  ]
  if with_cnt:
    out_type.append(jax.ShapeDtypeStruct((NC, N_PAD, CNTW), jnp.float32))
    scratch += [
        pltpu.VMEM((CHUNK, CNTW), jnp.float32),
        pltpu.VMEM((CHUNK, CNTW), jnp.float32),
        pltpu.VMEM_SHARED((N_PAD, CNTW), jnp.float32),
    ]

  def body(*refs):
    if with_cnt:
      (table, srcw, dstw, ones_hbm, out, cnt_out, idx_s, idx_d,
       rows0, rows1, acc_sh, sem0, sem1, ones_v, zbuf, cnt_sh) = refs
    else:
      (table, srcw, dstw, out,
       idx_s, idx_d, rows0, rows1, acc_sh, sem0, sem1) = refs
    c = lax.axis_index("c")
    s = lax.axis_index("s")
    slab = pl.ds(s * SLAB, SLAB)

    pltpu.sync_copy(srcw.at[c, s], idx_s)
    pltpu.sync_copy(dstw.at[s], idx_d)
    _zero_acc(rows0, acc_sh, s)
    extra = None
    if with_cnt:
      pltpu.sync_copy(ones_hbm, ones_v)
      zc = jnp.zeros((16,), jnp.float32)

      def czero(i, carry):
        zbuf[i, pl.ds(0, 16)] = zc
        return carry

      lax.fori_loop(0, CHUNK, czero, 0)
      for t in range(SLAB // CHUNK):
        pltpu.sync_copy(zbuf, cnt_sh.at[pl.ds(s * SLAB + t * CHUNK, CHUNK)])

      def extra(g):
        pltpu.sync_copy(ones_v, cnt_sh.at[idx_d.at[g]], add=True)

    plsc.subcore_barrier()

    _pipeline(table, idx_s, idx_d, rows0, rows1, acc_sh, sem0, sem1, NCH_A,
              extra=extra)

    plsc.subcore_barrier()
    pltpu.sync_copy(acc_sh.at[slab], out.at[c, slab])
    if with_cnt:
      pltpu.sync_copy(cnt_sh.at[slab], cnt_out.at[c, slab])

  return pl.kernel(body, out_type=out_type, mesh=_MESH, scratch_types=scratch,
                   compiler_params=_SC_PARAMS,
                   name="seg_colsplit" + ("_cnt" if with_cnt else ""))


def _make_seg_rowsplit():
  out_type = [jax.ShapeDtypeStruct((NC, N_PAD, DC), jnp.float32)]
  scratch = [
      pltpu.VMEM((NCH_B, CHUNK), jnp.int32),
      pltpu.VMEM((NCH_B, CHUNK), jnp.int32),
      pltpu.VMEM((CHUNK, DC), jnp.float32),
      pltpu.VMEM((CHUNK, DC), jnp.float32),
      pltpu.VMEM_SHARED((N_PAD, DC), jnp.float32),
      pltpu.SemaphoreType.DMA,
      pltpu.SemaphoreType.DMA,
  ]

  def body(table, srcw, dstw, out, idx_s, idx_d, rows0, rows1,
           acc_sh, sem0, sem1):
    c = lax.axis_index("c")
    s = lax.axis_index("s")
    wid = c * NS + s
    slab = pl.ds(s * SLAB, SLAB)

    pltpu.sync_copy(srcw.at[wid], idx_s)
    pltpu.sync_copy(dstw.at[wid], idx_d)
    _zero_acc(rows0, acc_sh, s)
    plsc.subcore_barrier()

    _pipeline(table, idx_s, idx_d, rows0, rows1, acc_sh, sem0, sem1, NCH_B)

    plsc.subcore_barrier()
    pltpu.sync_copy(acc_sh.at[slab], out.at[c, slab])

  return pl.kernel(body, out_type=out_type, mesh=_MESH, scratch_types=scratch,
                   compiler_params=_SC_PARAMS, name="seg_rowsplit")


_seg_col_cnt = _make_seg_colsplit(True)
_seg_col = _make_seg_colsplit(False)
_seg_row = _make_seg_rowsplit()


def _inv_deg(cntp_ref):
  cnt = cntp_ref[0, :, 0:1]
  return 1.0 / jnp.maximum(cnt, 1.0)


def _dot_t(a, w):
  return lax.dot_general(a, w, (((1,), (1,)), ((), ())),
                         preferred_element_type=jnp.float32)


_NB = 1000


def _tc1_body(aggp, cntp, x, w1l, b1l, w1r, h1):
  agg = jnp.concatenate([aggp[0], aggp[1]], axis=1) * _inv_deg(cntp)
  h = _dot_t(agg, w1l[...]) + b1l[...] + _dot_t(x[...], w1r[...])
  h1[...] = jnp.maximum(h, 0.0)


def _tc2_body(aggp, cntp, h1, w2l, b2l, w2r, w3l, h2, z):
  agg = jnp.concatenate([aggp[0], aggp[1]], axis=1) * _inv_deg(cntp)
  h = _dot_t(agg, w2l[...]) + b2l[...] + _dot_t(h1[...], w2r[...])
  h = jnp.maximum(h, 0.0)
  h2[...] = h
  z[...] = _dot_t(h, w3l[...])


def _tc3_body(aggp, cntp, h2, w3r, b3l, out):
  agg = (aggp[0] + aggp[1]) * _inv_deg(cntp)
  out[...] = agg + b3l[...] + _dot_t(h2[...], w3r[...])


def _row_spec(d):
  return pl.BlockSpec((_NB, d), lambda i: (i, 0))


def _part_spec(d):
  return pl.BlockSpec((NC, _NB, d), lambda i: (0, i, 0))


def _full_spec(shape):
  return pl.BlockSpec(shape, lambda i: tuple(0 for _ in shape))


_GRID = N_NODES // _NB

_tc1 = pl.pallas_call(
    _tc1_body,
    grid=(_GRID,),
    in_specs=[_part_spec(DC), _part_spec(CNTW), _row_spec(128),
              _full_spec((128, 128)), _full_spec((1, 128)),
              _full_spec((128, 128))],
    out_specs=_row_spec(128),
    out_shape=jax.ShapeDtypeStruct((N_NODES, 128), jnp.float32),
)

_tc2 = pl.pallas_call(
    _tc2_body,
    grid=(_GRID,),
    in_specs=[_part_spec(DC), _part_spec(CNTW), _row_spec(128),
              _full_spec((256, 128)), _full_spec((1, 256)),
              _full_spec((256, 128)), _full_spec((64, 256))],
    out_specs=[_row_spec(256), _row_spec(64)],
    out_shape=[jax.ShapeDtypeStruct((N_NODES, 256), jnp.float32),
               jax.ShapeDtypeStruct((N_NODES, 64), jnp.float32)],
)

_tc3 = pl.pallas_call(
    _tc3_body,
    grid=(_GRID,),
    in_specs=[_part_spec(64), _part_spec(CNTW), _row_spec(256),
              _full_spec((64, 256)), _full_spec((1, 64))],
    out_specs=_row_spec(64),
    out_shape=jax.ShapeDtypeStruct((N_NODES, 64), jnp.float32),
)


@jax.jit
def kernel(x, edge_index, W1l, b1l, W1r, W2l, b2l, W2r, W3l, b3l, W3r):
  src = edge_index[0].astype(jnp.int32)
  dst = edge_index[1].astype(jnp.int32)
  npad = E_PAD - N_EDGES
  srcp = jnp.concatenate([src, jnp.zeros((npad,), jnp.int32)])
  pad_dst = N_NODES + (jnp.arange(npad, dtype=jnp.int32) % (N_PAD - N_NODES))
  dstp = jnp.concatenate([dst, pad_dst])
  srcA = jnp.stack([2 * srcp, 2 * srcp + 1]).reshape(NC, NS, NCH_A, CHUNK)
  dstA = dstp.reshape(NS, NCH_A, CHUNK)
  srcB = srcp.reshape(NW, NCH_B, CHUNK)
  dstB = dstp.reshape(NW, NCH_B, CHUNK)
  ones = jnp.ones((CHUNK, CNTW), jnp.float32)

  x2 = x.reshape(2 * N_NODES, DC)
  aggp1, cntp = _seg_col_cnt(x2, srcA, dstA, ones)
  h1 = _tc1(aggp1, cntp, x, W1l, b1l.reshape(1, -1), W1r)
  aggp2, = _seg_col(h1.reshape(2 * N_NODES, DC), srcA, dstA)
  h2, z = _tc2(aggp2, cntp, h1, W2l, b2l.reshape(1, -1), W2r, W3l)
  aggp3, = _seg_row(z, srcB, dstB)
  return _tc3(aggp3, cntp, h2, W3r, b3l.reshape(1, -1))


# speedup vs baseline: 2.2116x; 2.1873x over previous
import jax
import jax.numpy as jnp
from jax import lax
from jax.experimental import pallas as pl
from jax.experimental.pallas import tpu as pltpu
from jax.experimental.pallas import tpu_sc as plsc

N_NODES = 10000
N_EDGES = 320000
NC, NS = 2, 16
NW = NC * NS
CHUNK = 80
E_PAD = 320000
NCH_A = E_PAD // NS // CHUNK
NCH_B = E_PAD // NW // CHUNK
N_PAD = 10240
SLAB = N_PAD // NS
CNTW = 16
DC = 64

_MESH = plsc.VectorSubcoreMesh(
    core_axis_name="c", subcore_axis_name="s", num_cores=NC, num_subcores=NS)
_SC_PARAMS = pltpu.CompilerParams(use_tc_tiling_on_sc=False)


def _pipeline(table, idx_s, idx_d, rows0, rows1, acc_sh, sem0, sem1, nchunk,
              extra=None):
  def gstart(g, buf, sem):
    pltpu.async_copy(table.at[idx_s.at[g]], buf, sem)

  def gwait(buf, sem):
    pltpu.make_async_copy(table.at[idx_s.at[0]], buf, sem).wait()

  def scat(g, buf):
    pltpu.sync_copy(buf, acc_sh.at[idx_d.at[g]], add=True)
    if extra is not None:
      extra(g)

  gstart(0, rows0, sem0)

  def step(g, carry):
    gstart(2 * g + 1, rows1, sem1)
    gwait(rows0, sem0)
    scat(2 * g, rows0)
    gstart(2 * g + 2, rows0, sem0)
    gwait(rows1, sem1)
    scat(2 * g + 1, rows1)
    return carry

  if nchunk % 2:
    lax.fori_loop(0, (nchunk - 1) // 2, step, 0)
    gwait(rows0, sem0)
    scat(nchunk - 1, rows0)
  else:
    lax.fori_loop(0, nchunk // 2 - 1, step, 0)
    gstart(nchunk - 1, rows1, sem1)
    gwait(rows0, sem0)
    scat(nchunk - 2, rows0)
    gwait(rows1, sem1)
    scat(nchunk - 1, rows1)


def _zero_acc(rows0, acc_sh, s):
  zv = jnp.zeros((16,), jnp.float32)
  vpr = DC // 16
  shift = vpr.bit_length() - 1

  def zstore(i, carry):
    rows0[i >> shift, pl.ds((i & (vpr - 1)) * 16, 16)] = zv
    return carry

  lax.fori_loop(0, CHUNK * vpr, zstore, 0)
  for t in range(SLAB // CHUNK):
    pltpu.sync_copy(rows0, acc_sh.at[pl.ds(s * SLAB + t * CHUNK, CHUNK)])


def _make_seg_colsplit(with_cnt):
  out_type = [jax.ShapeDtypeStruct((NC, N_PAD, DC), jnp.float32)]
  scratch = [
      pltpu.VMEM((NCH_A, CHUNK), jnp.int32),
      pltpu.VMEM((NCH_A, CHUNK), jnp.int32),
      pltpu.VMEM((CHUNK, DC), jnp.float32),
      pltpu.VMEM((CHUNK, DC), jnp.float32),
      pltpu.VMEM_SHARED((N_PAD, DC), jnp.float32),
      pltpu.SemaphoreType.DMA,
      pltpu.SemaphoreType.DMA,
  ]
  if with_cnt:
    out_type.append(jax.ShapeDtypeStruct((NC, N_PAD, CNTW), jnp.float32))
    scratch += [
        pltpu.VMEM((CHUNK, CNTW), jnp.float32),
        pltpu.VMEM((CHUNK, CNTW), jnp.float32),
        pltpu.VMEM_SHARED((N_PAD, CNTW), jnp.float32),
    ]

  def body(*refs):
    if with_cnt:
      (table, srcw, dstw, ones_hbm, out, cnt_out, idx_s, idx_d,
       rows0, rows1, acc_sh, sem0, sem1, ones_v, zbuf, cnt_sh) = refs
    else:
      (table, srcw, dstw, out,
       idx_s, idx_d, rows0, rows1, acc_sh, sem0, sem1) = refs
    c = lax.axis_index("c")
    s = lax.axis_index("s")
    slab = pl.ds(s * SLAB, SLAB)

    pltpu.sync_copy(srcw.at[c, s], idx_s)
    pltpu.sync_copy(dstw.at[s], idx_d)
    _zero_acc(rows0, acc_sh, s)
    extra = None
    if with_cnt:
      pltpu.sync_copy(ones_hbm, ones_v)
      zc = jnp.zeros((16,), jnp.float32)

      def czero(i, carry):
        zbuf[i, pl.ds(0, 16)] = zc
        return carry

      lax.fori_loop(0, CHUNK, czero, 0)
      for t in range(SLAB // CHUNK):
        pltpu.sync_copy(zbuf, cnt_sh.at[pl.ds(s * SLAB + t * CHUNK, CHUNK)])

      def extra(g):
        pltpu.sync_copy(ones_v, cnt_sh.at[idx_d.at[g]], add=True)

    plsc.subcore_barrier()

    _pipeline(table, idx_s, idx_d, rows0, rows1, acc_sh, sem0, sem1, NCH_A,
              extra=extra)

    plsc.subcore_barrier()
    pltpu.sync_copy(acc_sh.at[slab], out.at[c, slab])
    if with_cnt:
      pltpu.sync_copy(cnt_sh.at[slab], cnt_out.at[c, slab])

  return pl.kernel(body, out_type=out_type, mesh=_MESH, scratch_types=scratch,
                   compiler_params=_SC_PARAMS,
                   name="seg_colsplit" + ("_cnt" if with_cnt else ""))


def _make_seg_rowsplit():
  out_type = [jax.ShapeDtypeStruct((NC, N_PAD, DC), jnp.float32)]
  scratch = [
      pltpu.VMEM((NCH_B, CHUNK), jnp.int32),
      pltpu.VMEM((NCH_B, CHUNK), jnp.int32),
      pltpu.VMEM((CHUNK, DC), jnp.float32),
      pltpu.VMEM((CHUNK, DC), jnp.float32),
      pltpu.VMEM_SHARED((N_PAD, DC), jnp.float32),
      pltpu.SemaphoreType.DMA,
      pltpu.SemaphoreType.DMA,
  ]

  def body(table, srcw, dstw, out, idx_s, idx_d, rows0, rows1,
           acc_sh, sem0, sem1):
    c = lax.axis_index("c")
    s = lax.axis_index("s")
    wid = c * NS + s
    slab = pl.ds(s * SLAB, SLAB)

    pltpu.sync_copy(srcw.at[wid], idx_s)
    pltpu.sync_copy(dstw.at[wid], idx_d)
    _zero_acc(rows0, acc_sh, s)
    plsc.subcore_barrier()

    _pipeline(table, idx_s, idx_d, rows0, rows1, acc_sh, sem0, sem1, NCH_B)

    plsc.subcore_barrier()
    pltpu.sync_copy(acc_sh.at[slab], out.at[c, slab])

  return pl.kernel(body, out_type=out_type, mesh=_MESH, scratch_types=scratch,
                   compiler_params=_SC_PARAMS, name="seg_rowsplit")


_seg_col_cnt = _make_seg_colsplit(True)
_seg_col = _make_seg_colsplit(False)
_seg_row = _make_seg_rowsplit()


def _inv_deg(cntp_ref):
  cnt = cntp_ref[0, :, 0:1]
  return 1.0 / jnp.maximum(cnt, 1.0)


def _dot_t(a, w):
  return lax.dot_general(a, w, (((1,), (1,)), ((), ())),
                         preferred_element_type=jnp.float32)


_NB = 1000


def _tc1_body(aggp, cntp, x, w1l, b1l, w1r, h1):
  agg = jnp.concatenate([aggp[0], aggp[1]], axis=1) * _inv_deg(cntp)
  h = _dot_t(agg, w1l[...]) + b1l[...] + _dot_t(x[...], w1r[...])
  h1[...] = jnp.maximum(h, 0.0)


def _tc2_body(aggp, cntp, h1, w2l, b2l, w2r, w3l, h2, z):
  agg = jnp.concatenate([aggp[0], aggp[1]], axis=1) * _inv_deg(cntp)
  h = _dot_t(agg, w2l[...]) + b2l[...] + _dot_t(h1[...], w2r[...])
  h = jnp.maximum(h, 0.0)
  h2[...] = h
  z[...] = _dot_t(h, w3l[...])


def _tc3_body(aggp, cntp, h2, w3r, b3l, out):
  agg = (aggp[0] + aggp[1]) * _inv_deg(cntp)
  out[...] = agg + b3l[...] + _dot_t(h2[...], w3r[...])


def _row_spec(d):
  return pl.BlockSpec((_NB, d), lambda i: (i, 0))


def _part_spec(d):
  return pl.BlockSpec((NC, _NB, d), lambda i: (0, i, 0))


def _full_spec(shape):
  return pl.BlockSpec(shape, lambda i: tuple(0 for _ in shape))


_GRID = N_NODES // _NB

_tc1 = pl.pallas_call(
    _tc1_body,
    grid=(_GRID,),
    in_specs=[_part_spec(DC), _part_spec(CNTW), _row_spec(128),
              _full_spec((128, 128)), _full_spec((1, 128)),
              _full_spec((128, 128))],
    out_specs=_row_spec(128),
    out_shape=jax.ShapeDtypeStruct((N_NODES, 128), jnp.float32),
)

_tc2 = pl.pallas_call(
    _tc2_body,
    grid=(_GRID,),
    in_specs=[_part_spec(DC), _part_spec(CNTW), _row_spec(128),
              _full_spec((256, 128)), _full_spec((1, 256)),
              _full_spec((256, 128)), _full_spec((64, 256))],
    out_specs=[_row_spec(256), _row_spec(64)],
    out_shape=[jax.ShapeDtypeStruct((N_NODES, 256), jnp.float32),
               jax.ShapeDtypeStruct((N_NODES, 64), jnp.float32)],
)

_tc3 = pl.pallas_call(
    _tc3_body,
    grid=(_GRID,),
    in_specs=[_part_spec(64), _part_spec(CNTW), _row_spec(256),
              _full_spec((64, 256)), _full_spec((1, 64))],
    out_specs=_row_spec(64),
    out_shape=jax.ShapeDtypeStruct((N_NODES, 64), jnp.float32),
)


@jax.jit
def kernel(x, edge_index, W1l, b1l, W1r, W2l, b2l, W2r, W3l, b3l, W3r):
  src = edge_index[0].astype(jnp.int32)
  dst = edge_index[1].astype(jnp.int32)
  npad = E_PAD - N_EDGES
  srcp = jnp.concatenate([src, jnp.zeros((npad,), jnp.int32)])
  pad_dst = N_NODES + (jnp.arange(npad, dtype=jnp.int32) % (N_PAD - N_NODES))
  dstp = jnp.concatenate([dst, pad_dst])
  srcA = jnp.stack([2 * srcp, 2 * srcp + 1]).reshape(NC, NS, NCH_A, CHUNK)
  dstA = dstp.reshape(NS, NCH_A, CHUNK)
  srcB = srcp.reshape(NW, NCH_B, CHUNK)
  dstB = dstp.reshape(NW, NCH_B, CHUNK)
  ones = jnp.ones((CHUNK, CNTW), jnp.float32)

  x2 = x.reshape(2 * N_NODES, DC)
  aggp1, cntp = _seg_col_cnt(x2, srcA, dstA, ones)
  h1 = _tc1(aggp1, cntp, x, W1l, b1l.reshape(1, -1), W1r)
  aggp2, = _seg_col(h1.reshape(2 * N_NODES, DC), srcA, dstA)
  h2, z = _tc2(aggp2, cntp, h1, W2l, b2l.reshape(1, -1), W2r, W3l)
  aggp3, = _seg_row(z, srcB, dstB)
  return _tc3(aggp3, cntp, h2, W3r, b3l.reshape(1, -1))


# speedup vs baseline: 2.4439x; 1.1050x over previous
import jax
import jax.numpy as jnp
from jax import lax
from jax.experimental import pallas as pl
from jax.experimental.pallas import tpu as pltpu
from jax.experimental.pallas import tpu_sc as plsc

N_NODES = 10000
N_EDGES = 320000
NC, NS = 2, 16
NW = NC * NS
CHUNK = 80
E_PAD = 320000
NCH_A = E_PAD // NS // CHUNK
NCH_B = E_PAD // NW // CHUNK
N_PAD = 10240
SLAB = N_PAD // NS
CNTW = 16
DC = 64

_MESH = plsc.VectorSubcoreMesh(
    core_axis_name="c", subcore_axis_name="s", num_cores=NC, num_subcores=NS)
_SC_PARAMS = pltpu.CompilerParams(use_tc_tiling_on_sc=False)


def _pipeline(table, idx_s, idx_d, bufs, acc_sh, gsems, ssems, nchunk,
              extra=None):
  def gstart(g, b):
    pltpu.async_copy(table.at[idx_s.at[g]], bufs[b], gsems[b])

  def gwait(b):
    pltpu.make_async_copy(table.at[idx_s.at[0]], bufs[b], gsems[b]).wait()

  def sstart(g, b):
    pltpu.async_copy(bufs[b], acc_sh.at[idx_d.at[g]], ssems[b], add=True)
    if extra is not None:
      extra(g)

  def swait(b):
    pltpu.make_async_copy(bufs[b], acc_sh.at[idx_d.at[0]], ssems[b]).wait()

  assert nchunk >= 6
  gstart(0, 0)
  gstart(1, 1)
  gwait(0); sstart(0, 0); gstart(2, 2)
  gwait(1); sstart(1, 1); gstart(3, 3)
  gwait(2); sstart(2, 2); swait(0); gstart(4, 0)
  gwait(3); sstart(3, 3); swait(1); gstart(5, 1)
  pend = {2, 3}

  kq = (nchunk - 6) // 4

  def quad(k, carry):
    base = 4 + 4 * k
    for j in range(4):
      gwait(j)
      sstart(base + j, j)
      swait((j + 2) % 4)
      gstart(base + j + 2, (j + 2) % 4)
    return carry

  lax.fori_loop(0, kq, quad, 0)

  base_end = 4 + 4 * kq
  for g in range(base_end, nchunk):
    b = g % 4
    gwait(b)
    sstart(g, b)
    pend.add(b)
    nxt = g + 2
    if nxt < nchunk:
      swait(nxt % 4)
      pend.discard(nxt % 4)
      gstart(nxt, nxt % 4)
  for b in sorted(pend):
    swait(b)


def _zero_acc(rows0, acc_sh, s):
  zv = jnp.zeros((16,), jnp.float32)
  vpr = DC // 16
  shift = vpr.bit_length() - 1

  def zstore(i, carry):
    rows0[i >> shift, pl.ds((i & (vpr - 1)) * 16, 16)] = zv
    return carry

  lax.fori_loop(0, CHUNK * vpr, zstore, 0)
  for t in range(SLAB // CHUNK):
    pltpu.sync_copy(rows0, acc_sh.at[pl.ds(s * SLAB + t * CHUNK, CHUNK)])


def _make_seg_colsplit(with_cnt):
  out_type = [jax.ShapeDtypeStruct((NC, N_PAD, DC), jnp.float32)]
  scratch = [
      pltpu.VMEM((NCH_A, CHUNK), jnp.int32),
      pltpu.VMEM((NCH_A, CHUNK), jnp.int32),
      [pltpu.VMEM((CHUNK, DC), jnp.float32)] * 4,
      pltpu.VMEM_SHARED((N_PAD, DC), jnp.float32),
      [pltpu.SemaphoreType.DMA] * 4,
      [pltpu.SemaphoreType.DMA] * 4,
  ]
  if with_cnt:
    out_type.append(jax.ShapeDtypeStruct((NC, N_PAD, CNTW), jnp.float32))
    scratch += [
        pltpu.VMEM((CHUNK, CNTW), jnp.float32),
        pltpu.VMEM((CHUNK, CNTW), jnp.float32),
        pltpu.VMEM_SHARED((N_PAD, CNTW), jnp.float32),
    ]

  def body(*refs):
    if with_cnt:
      (table, srcw, dstw, ones_hbm, out, cnt_out, idx_s, idx_d,
       bufs, acc_sh, gsems, ssems, ones_v, zbuf, cnt_sh) = refs
    else:
      (table, srcw, dstw, out,
       idx_s, idx_d, bufs, acc_sh, gsems, ssems) = refs
    c = lax.axis_index("c")
    s = lax.axis_index("s")
    slab = pl.ds(s * SLAB, SLAB)

    pltpu.sync_copy(srcw.at[c, s], idx_s)
    pltpu.sync_copy(dstw.at[s], idx_d)
    _zero_acc(bufs[0], acc_sh, s)
    extra = None
    if with_cnt:
      pltpu.sync_copy(ones_hbm, ones_v)
      zc = jnp.zeros((16,), jnp.float32)

      def czero(i, carry):
        zbuf[i, pl.ds(0, 16)] = zc
        return carry

      lax.fori_loop(0, CHUNK, czero, 0)
      for t in range(SLAB // CHUNK):
        pltpu.sync_copy(zbuf, cnt_sh.at[pl.ds(s * SLAB + t * CHUNK, CHUNK)])

      def extra(g):
        pltpu.sync_copy(ones_v, cnt_sh.at[idx_d.at[g]], add=True)

    plsc.subcore_barrier()

    _pipeline(table, idx_s, idx_d, bufs, acc_sh, gsems, ssems, NCH_A,
              extra=extra)

    plsc.subcore_barrier()
    pltpu.sync_copy(acc_sh.at[slab], out.at[c, slab])
    if with_cnt:
      pltpu.sync_copy(cnt_sh.at[slab], cnt_out.at[c, slab])

  return pl.kernel(body, out_type=out_type, mesh=_MESH, scratch_types=scratch,
                   compiler_params=_SC_PARAMS,
                   name="seg_colsplit" + ("_cnt" if with_cnt else ""))


def _make_seg_rowsplit():
  out_type = [jax.ShapeDtypeStruct((NC, N_PAD, DC), jnp.float32)]
  scratch = [
      pltpu.VMEM((NCH_B, CHUNK), jnp.int32),
      pltpu.VMEM((NCH_B, CHUNK), jnp.int32),
      [pltpu.VMEM((CHUNK, DC), jnp.float32)] * 4,
      pltpu.VMEM_SHARED((N_PAD, DC), jnp.float32),
      [pltpu.SemaphoreType.DMA] * 4,
      [pltpu.SemaphoreType.DMA] * 4,
  ]

  def body(table, srcw, dstw, out, idx_s, idx_d, bufs,
           acc_sh, gsems, ssems):
    c = lax.axis_index("c")
    s = lax.axis_index("s")
    wid = c * NS + s
    slab = pl.ds(s * SLAB, SLAB)

    pltpu.sync_copy(srcw.at[wid], idx_s)
    pltpu.sync_copy(dstw.at[wid], idx_d)
    _zero_acc(bufs[0], acc_sh, s)
    plsc.subcore_barrier()

    _pipeline(table, idx_s, idx_d, bufs, acc_sh, gsems, ssems, NCH_B)

    plsc.subcore_barrier()
    pltpu.sync_copy(acc_sh.at[slab], out.at[c, slab])

  return pl.kernel(body, out_type=out_type, mesh=_MESH, scratch_types=scratch,
                   compiler_params=_SC_PARAMS, name="seg_rowsplit")


_seg_col_cnt = _make_seg_colsplit(True)
_seg_col = _make_seg_colsplit(False)
_seg_row = _make_seg_rowsplit()


def _inv_deg(cntp_ref):
  cnt = cntp_ref[0, :, 0:1]
  return 1.0 / jnp.maximum(cnt, 1.0)


def _dot_t(a, w):
  return lax.dot_general(a, w, (((1,), (1,)), ((), ())),
                         preferred_element_type=jnp.float32)


_NB = 1000


def _tc1_body(aggp, cntp, x, w1l, b1l, w1r, h1):
  agg = jnp.concatenate([aggp[0], aggp[1]], axis=1) * _inv_deg(cntp)
  h = _dot_t(agg, w1l[...]) + b1l[...] + _dot_t(x[...], w1r[...])
  h1[...] = jnp.maximum(h, 0.0)


def _tc2_body(aggp, cntp, h1, w2l, b2l, w2r, w3l, h2, z):
  agg = jnp.concatenate([aggp[0], aggp[1]], axis=1) * _inv_deg(cntp)
  h = _dot_t(agg, w2l[...]) + b2l[...] + _dot_t(h1[...], w2r[...])
  h = jnp.maximum(h, 0.0)
  h2[...] = h
  z[...] = _dot_t(h, w3l[...])


def _tc3_body(aggp, cntp, h2, w3r, b3l, out):
  agg = (aggp[0] + aggp[1]) * _inv_deg(cntp)
  out[...] = agg + b3l[...] + _dot_t(h2[...], w3r[...])


def _row_spec(d):
  return pl.BlockSpec((_NB, d), lambda i: (i, 0))


def _part_spec(d):
  return pl.BlockSpec((NC, _NB, d), lambda i: (0, i, 0))


def _full_spec(shape):
  return pl.BlockSpec(shape, lambda i: tuple(0 for _ in shape))


_GRID = N_NODES // _NB

_tc1 = pl.pallas_call(
    _tc1_body,
    grid=(_GRID,),
    in_specs=[_part_spec(DC), _part_spec(CNTW), _row_spec(128),
              _full_spec((128, 128)), _full_spec((1, 128)),
              _full_spec((128, 128))],
    out_specs=_row_spec(128),
    out_shape=jax.ShapeDtypeStruct((N_NODES, 128), jnp.float32),
)

_tc2 = pl.pallas_call(
    _tc2_body,
    grid=(_GRID,),
    in_specs=[_part_spec(DC), _part_spec(CNTW), _row_spec(128),
              _full_spec((256, 128)), _full_spec((1, 256)),
              _full_spec((256, 128)), _full_spec((64, 256))],
    out_specs=[_row_spec(256), _row_spec(64)],
    out_shape=[jax.ShapeDtypeStruct((N_NODES, 256), jnp.float32),
               jax.ShapeDtypeStruct((N_NODES, 64), jnp.float32)],
)

_tc3 = pl.pallas_call(
    _tc3_body,
    grid=(_GRID,),
    in_specs=[_part_spec(64), _part_spec(CNTW), _row_spec(256),
              _full_spec((64, 256)), _full_spec((1, 64))],
    out_specs=_row_spec(64),
    out_shape=jax.ShapeDtypeStruct((N_NODES, 64), jnp.float32),
)


@jax.jit
def kernel(x, edge_index, W1l, b1l, W1r, W2l, b2l, W2r, W3l, b3l, W3r):
  src = edge_index[0].astype(jnp.int32)
  dst = edge_index[1].astype(jnp.int32)
  npad = E_PAD - N_EDGES
  srcp = jnp.concatenate([src, jnp.zeros((npad,), jnp.int32)])
  pad_dst = N_NODES + (jnp.arange(npad, dtype=jnp.int32) % (N_PAD - N_NODES))
  dstp = jnp.concatenate([dst, pad_dst])
  srcA = jnp.stack([2 * srcp, 2 * srcp + 1]).reshape(NC, NS, NCH_A, CHUNK)
  dstA = dstp.reshape(NS, NCH_A, CHUNK)
  srcB = srcp.reshape(NW, NCH_B, CHUNK)
  dstB = dstp.reshape(NW, NCH_B, CHUNK)
  ones = jnp.ones((CHUNK, CNTW), jnp.float32)

  x2 = x.reshape(2 * N_NODES, DC)
  aggp1, cntp = _seg_col_cnt(x2, srcA, dstA, ones)
  h1 = _tc1(aggp1, cntp, x, W1l, b1l.reshape(1, -1), W1r)
  aggp2, = _seg_col(h1.reshape(2 * N_NODES, DC), srcA, dstA)
  h2, z = _tc2(aggp2, cntp, h1, W2l, b2l.reshape(1, -1), W2r, W3l)
  aggp3, = _seg_row(z, srcB, dstB)
  return _tc3(aggp3, cntp, h2, W3r, b3l.reshape(1, -1))
